# baseline (speedup 1.0000x reference)
import functools

import jax
import jax.numpy as jnp
from jax import lax
from jax.experimental import pallas as pl
from jax.experimental.pallas import tpu as pltpu
from jax.experimental.pallas import tpu_sc as plsc

_NB = 256
_BINS = _NB * _NB
_N = 8388608

_NC = 2
_NS = 16
_NW = _NC * _NS
_PER_W = _N // _NW
_CH = 8192
_NCHUNK = _PER_W // _CH

_mesh = plsc.VectorSubcoreMesh(core_axis_name="c", subcore_axis_name="s")


@functools.partial(
    pl.kernel,
    out_type=jax.ShapeDtypeStruct((_NW, _NB, _NB), jnp.float32),
    mesh=_mesh,
    scratch_types=[
        pltpu.VMEM((_NB, _NB), jnp.float32),
        pltpu.VMEM((_CH,), jnp.int32),
        pltpu.VMEM((_CH,), jnp.int32),
        pltpu.VMEM((_CH,), jnp.int32),
        pltpu.VMEM((_CH,), jnp.int32),
        pltpu.SemaphoreType.DMA,
        pltpu.SemaphoreType.DMA,
        pltpu.SemaphoreType.DMA,
        pltpu.SemaphoreType.DMA,
    ],
    compiler_params=pltpu.CompilerParams(needs_layout_passes=False),
)
def _joint_hist(g_hbm, i_hbm, out_hbm, hist, g0, g1, i0, i1,
                sg0, sg1, si0, si1):
    c = lax.axis_index("c")
    s = lax.axis_index("s")
    wid = s * _NC + c
    base = wid * _PER_W

    gb = [g0, g1]
    ib = [i0, i1]
    gsem = [sg0, sg1]
    isem = [si0, si1]
    ones = jnp.full((16,), 1.0, jnp.float32)
    zero16 = jnp.zeros((16,), jnp.float32)
    iota16 = lax.iota(jnp.int32, 16)

    for b in range(2):
        off = base + b * _CH
        pltpu.async_copy(g_hbm.at[pl.ds(off, _CH)], gb[b], gsem[b])
        pltpu.async_copy(i_hbm.at[pl.ds(off, _CH)], ib[b], isem[b])

    @pl.loop(0, _NB)
    def _(r):
        for q in range(_NB // 16):
            hist[r, pl.ds(q * 16, 16)] = zero16

    @pl.loop(0, _NCHUNK, step=2)
    def _(chunk0):
        for b in range(2):
            chunk = chunk0 + b
            pltpu.make_async_copy(g_hbm.at[pl.ds(0, _CH)], gb[b], gsem[b]).wait()
            pltpu.make_async_copy(i_hbm.at[pl.ds(0, _CH)], ib[b], isem[b]).wait()

            @plsc.parallel_loop(0, _CH // 16, unroll=8)
            def _(j):
                gv = gb[b][pl.ds(j * 16, 16)]
                iv = ib[b][pl.ds(j * 16, 16)]
                plsc.addupdate_scatter(hist, [gv, iv], ones)

            @pl.when(chunk + 2 < _NCHUNK)
            def _():
                off = base + (chunk + 2) * _CH
                pltpu.async_copy(g_hbm.at[pl.ds(off, _CH)], gb[b], gsem[b])
                pltpu.async_copy(i_hbm.at[pl.ds(off, _CH)], ib[b], isem[b])

    pltpu.sync_copy(hist, out_hbm.at[wid])


_TCB = 4


def _mi_body(parts_ref, out_ref, acc_ref):
    w = pl.program_id(0)
    blk = parts_ref[0]
    for t in range(1, _TCB):
        blk = blk + parts_ref[t]

    @pl.when(w == 0)
    def _():
        acc_ref[...] = blk

    @pl.when(w > 0)
    def _():
        acc_ref[...] += blk

    @pl.when(w == _NW // _TCB - 1)
    def _():
        counts = acc_ref[...]
        joint = counts * (1.0 / _N)
        mg = jnp.sum(joint, axis=1, keepdims=True)
        mi = jnp.sum(joint, axis=0, keepdims=True)
        valid = counts > 0.0
        denom = jnp.where(valid, mg * mi, 1.0)
        ratio = jnp.where(valid, joint / denom, 1.0)
        terms = jnp.where(valid, joint * jnp.log(ratio), 0.0)
        out_ref[...] = jnp.sum(counts * terms, keepdims=True)


_mi_call = pl.pallas_call(
    _mi_body,
    out_shape=jax.ShapeDtypeStruct((1, 1), jnp.float32),
    grid=(_NW // _TCB,),
    in_specs=[pl.BlockSpec((_TCB, _NB, _NB), lambda w: (w, 0, 0))],
    out_specs=pl.BlockSpec((1, 1), lambda w: (0, 0)),
    scratch_shapes=[pltpu.VMEM((_NB, _NB), jnp.float32)],
)


def kernel(global_vector, infered_vector):
    parts = _joint_hist(global_vector, infered_vector)
    mi = _mi_call(parts)
    return mi[0, 0]


# speedup vs baseline: 1.0244x; 1.0244x over previous
import functools

import jax
import jax.numpy as jnp
from jax import lax
from jax.experimental import pallas as pl
from jax.experimental.pallas import tpu as pltpu
from jax.experimental.pallas import tpu_sc as plsc

_NB = 256
_BINS = _NB * _NB
_N = 8388608

_NC = 2
_NS = 16
_NW = _NC * _NS
_PER_W = _N // _NW
_CH = 8192
_NCHUNK = _PER_W // _CH

_mesh = plsc.VectorSubcoreMesh(core_axis_name="c", subcore_axis_name="s")


@functools.partial(
    pl.kernel,
    out_type=jax.ShapeDtypeStruct((_NW, _NB, _NB), jnp.float32),
    mesh=_mesh,
    scratch_types=[
        pltpu.VMEM((_NB, _NB), jnp.float32),
        pltpu.VMEM((_CH,), jnp.int32),
        pltpu.VMEM((_CH,), jnp.int32),
        pltpu.VMEM((_CH,), jnp.int32),
        pltpu.VMEM((_CH,), jnp.int32),
        pltpu.VMEM((_CH,), jnp.int32),
        pltpu.VMEM((_CH,), jnp.int32),
        pltpu.SemaphoreType.DMA,
        pltpu.SemaphoreType.DMA,
        pltpu.SemaphoreType.DMA,
        pltpu.SemaphoreType.DMA,
        pltpu.SemaphoreType.DMA,
        pltpu.SemaphoreType.DMA,
    ],
    compiler_params=pltpu.CompilerParams(needs_layout_passes=False),
)
def _joint_hist(g_hbm, i_hbm, out_hbm, hist, g0, g1, g2, i0, i1, i2,
                sg0, sg1, sg2, si0, si1, si2):
    c = lax.axis_index("c")
    s = lax.axis_index("s")
    wid = s * _NC + c
    base = wid * _PER_W

    nbuf = 3
    gb = [g0, g1, g2]
    ib = [i0, i1, i2]
    gsem = [sg0, sg1, sg2]
    isem = [si0, si1, si2]
    ones = jnp.full((16,), 1.0, jnp.float32)
    zero16 = jnp.zeros((16,), jnp.float32)

    for b in range(nbuf):
        off = base + b * _CH
        pltpu.async_copy(g_hbm.at[pl.ds(off, _CH)], gb[b], gsem[b])
        pltpu.async_copy(i_hbm.at[pl.ds(off, _CH)], ib[b], isem[b])

    @pl.loop(0, _NB)
    def _(r):
        for q in range(_NB // 16):
            hist[r, pl.ds(q * 16, 16)] = zero16

    def _do_chunk(chunk, b, refill):
        pltpu.make_async_copy(g_hbm.at[pl.ds(0, _CH)], gb[b], gsem[b]).wait()
        pltpu.make_async_copy(i_hbm.at[pl.ds(0, _CH)], ib[b], isem[b]).wait()

        @plsc.parallel_loop(0, _CH // 16, unroll=8)
        def _(j):
            gv = gb[b][pl.ds(j * 16, 16)]
            iv = ib[b][pl.ds(j * 16, 16)]
            plsc.addupdate_scatter(hist, [gv, iv], ones)

        if refill:
            @pl.when(chunk + nbuf < _NCHUNK)
            def _():
                off = base + (chunk + nbuf) * _CH
                pltpu.async_copy(g_hbm.at[pl.ds(off, _CH)], gb[b], gsem[b])
                pltpu.async_copy(i_hbm.at[pl.ds(off, _CH)], ib[b], isem[b])

    @pl.loop(0, _NCHUNK - nbuf, step=nbuf)
    def _(chunk0):
        for b in range(nbuf):
            _do_chunk(chunk0 + b, b, True)

    for b in range(nbuf - 1):
        _do_chunk(_NCHUNK - nbuf + b + 1, b, False)

    pltpu.sync_copy(hist, out_hbm.at[wid])


def _mi_body(parts_ref, out_ref):
    counts = parts_ref[0]
    for t in range(1, _NW):
        counts = counts + parts_ref[t]
    joint = counts * (1.0 / _N)
    mg = jnp.sum(joint, axis=1, keepdims=True)
    mi = jnp.sum(joint, axis=0, keepdims=True)
    valid = counts > 0.0
    denom = jnp.where(valid, mg * mi, 1.0)
    ratio = jnp.where(valid, joint / denom, 1.0)
    terms = jnp.where(valid, joint * jnp.log(ratio), 0.0)
    out_ref[...] = jnp.sum(counts * terms, keepdims=True)


_mi_call = pl.pallas_call(
    _mi_body,
    out_shape=jax.ShapeDtypeStruct((1, 1), jnp.float32),
)


def kernel(global_vector, infered_vector):
    parts = _joint_hist(global_vector, infered_vector)
    mi = _mi_call(parts)
    return mi[0, 0]


# speedup vs baseline: 1.0322x; 1.0076x over previous
import functools

import jax
import jax.numpy as jnp
from jax import lax
from jax.experimental import pallas as pl
from jax.experimental.pallas import tpu as pltpu
from jax.experimental.pallas import tpu_sc as plsc

_NB = 256
_BINS = _NB * _NB
_N = 8388608

_NC = 2
_NS = 16
_NW = _NC * _NS
_PER_W = _N // _NW
_CH = 8192
_NCHUNK = _PER_W // _CH

_mesh = plsc.VectorSubcoreMesh(core_axis_name="c", subcore_axis_name="s")


@functools.partial(
    pl.kernel,
    out_type=jax.ShapeDtypeStruct((_NW, _NB, _NB), jnp.float32),
    mesh=_mesh,
    scratch_types=[
        pltpu.VMEM((_NB, _NB), jnp.float32),
        pltpu.VMEM((_CH,), jnp.int32),
        pltpu.VMEM((_CH,), jnp.int32),
        pltpu.VMEM((_CH,), jnp.int32),
        pltpu.VMEM((_CH,), jnp.int32),
        pltpu.SemaphoreType.DMA,
        pltpu.SemaphoreType.DMA,
        pltpu.SemaphoreType.DMA,
        pltpu.SemaphoreType.DMA,
    ],
    compiler_params=pltpu.CompilerParams(needs_layout_passes=False),
)
def _joint_hist(g_hbm, i_hbm, out_hbm, hist, g0, g1, i0, i1,
                sg0, sg1, si0, si1):
    c = lax.axis_index("c")
    s = lax.axis_index("s")
    wid = s * _NC + c
    base = wid * _PER_W

    gb = [g0, g1]
    ib = [i0, i1]
    gsem = [sg0, sg1]
    isem = [si0, si1]
    ones = jnp.full((16,), 1.0, jnp.float32)
    zero16 = jnp.zeros((16,), jnp.float32)

    for b in range(2):
        off = base + b * _CH
        pltpu.async_copy(g_hbm.at[pl.ds(off, _CH)], gb[b], gsem[b])
        pltpu.async_copy(i_hbm.at[pl.ds(off, _CH)], ib[b], isem[b])

    @pl.loop(0, _NB)
    def _(r):
        for q in range(_NB // 16):
            hist[r, pl.ds(q * 16, 16)] = zero16

    @pl.loop(0, _NCHUNK, step=2)
    def _(chunk0):
        for b in range(2):
            chunk = chunk0 + b
            pltpu.make_async_copy(g_hbm.at[pl.ds(0, _CH)], gb[b], gsem[b]).wait()
            pltpu.make_async_copy(i_hbm.at[pl.ds(0, _CH)], ib[b], isem[b]).wait()

            @plsc.parallel_loop(0, _CH // 16, unroll=8)
            def _(j):
                gv = gb[b][pl.ds(j * 16, 16)]
                iv = ib[b][pl.ds(j * 16, 16)]
                plsc.addupdate_scatter(hist, [gv, iv], ones)

            @pl.when(chunk + 2 < _NCHUNK)
            def _():
                off = base + (chunk + 2) * _CH
                pltpu.async_copy(g_hbm.at[pl.ds(off, _CH)], gb[b], gsem[b])
                pltpu.async_copy(i_hbm.at[pl.ds(off, _CH)], ib[b], isem[b])

    pltpu.sync_copy(hist, out_hbm.at[wid])


def _mi_body(parts_ref, out_ref):
    counts = parts_ref[0]
    for t in range(1, _NW):
        counts = counts + parts_ref[t]
    joint = counts * (1.0 / _N)
    mg = jnp.sum(joint, axis=1, keepdims=True)
    mi = jnp.sum(joint, axis=0, keepdims=True)
    valid = counts > 0.0
    denom = jnp.where(valid, mg * mi, 1.0)
    ratio = jnp.where(valid, joint / denom, 1.0)
    terms = jnp.where(valid, joint * jnp.log(ratio), 0.0)
    out_ref[...] = jnp.sum(counts * terms, keepdims=True)


_mi_call = pl.pallas_call(
    _mi_body,
    out_shape=jax.ShapeDtypeStruct((1, 1), jnp.float32),
)


def kernel(global_vector, infered_vector):
    parts = _joint_hist(global_vector, infered_vector)
    mi = _mi_call(parts)
    return mi[0, 0]


# speedup vs baseline: 1.0338x; 1.0015x over previous
import functools

import jax
import jax.numpy as jnp
from jax import lax
from jax.experimental import pallas as pl
from jax.experimental.pallas import tpu as pltpu
from jax.experimental.pallas import tpu_sc as plsc

_NB = 256
_BINS = _NB * _NB
_N = 8388608

_NC = 2
_NS = 16
_NW = _NC * _NS
_PER_W = _N // _NW
_CH = 8192
_NCHUNK = _PER_W // _CH

_mesh = plsc.VectorSubcoreMesh(core_axis_name="c", subcore_axis_name="s",
                               num_cores=_NC, num_subcores=_NS)


@functools.partial(
    pl.kernel,
    out_type=jax.ShapeDtypeStruct((_NW, _NB, _NB), jnp.float32),
    mesh=_mesh,
    scratch_types=[
        pltpu.VMEM((_NB, _NB), jnp.float32),
        pltpu.VMEM((_CH,), jnp.int32),
        pltpu.VMEM((_CH,), jnp.int32),
        pltpu.VMEM((_CH,), jnp.int32),
        pltpu.VMEM((_CH,), jnp.int32),
        pltpu.SemaphoreType.DMA,
        pltpu.SemaphoreType.DMA,
        pltpu.SemaphoreType.DMA,
        pltpu.SemaphoreType.DMA,
    ],
    compiler_params=pltpu.CompilerParams(needs_layout_passes=False),
)
def _joint_hist(g_hbm, i_hbm, out_hbm, hist, g0, g1, i0, i1,
                sg0, sg1, si0, si1):
    c = lax.axis_index("c")
    s = lax.axis_index("s")
    wid = s * _NC + c
    base = wid * _PER_W

    gb = [g0, g1]
    ib = [i0, i1]
    gsem = [sg0, sg1]
    isem = [si0, si1]
    ones = jnp.full((16,), 1.0, jnp.float32)
    zero16 = jnp.zeros((16,), jnp.float32)

    for b in range(2):
        off = base + b * _CH
        pltpu.async_copy(g_hbm.at[pl.ds(off, _CH)], gb[b], gsem[b])
        pltpu.async_copy(i_hbm.at[pl.ds(off, _CH)], ib[b], isem[b])

    @pl.loop(0, _NB)
    def _(r):
        for q in range(_NB // 16):
            hist[r, pl.ds(q * 16, 16)] = zero16

    @pl.loop(0, _NCHUNK, step=2)
    def _(chunk0):
        for b in range(2):
            chunk = chunk0 + b
            pltpu.make_async_copy(g_hbm.at[pl.ds(0, _CH)], gb[b], gsem[b]).wait()
            pltpu.make_async_copy(i_hbm.at[pl.ds(0, _CH)], ib[b], isem[b]).wait()

            @plsc.parallel_loop(0, _CH // 16, unroll=8)
            def _(j):
                gv = gb[b][pl.ds(j * 16, 16)]
                iv = ib[b][pl.ds(j * 16, 16)]
                plsc.addupdate_scatter(hist, [gv, iv], ones)

            @pl.when(chunk + 2 < _NCHUNK)
            def _():
                off = base + (chunk + 2) * _CH
                pltpu.async_copy(g_hbm.at[pl.ds(off, _CH)], gb[b], gsem[b])
                pltpu.async_copy(i_hbm.at[pl.ds(off, _CH)], ib[b], isem[b])

    pltpu.sync_copy(hist, out_hbm.at[wid])


def _mi_body(parts_ref, out_ref):
    counts = parts_ref[0]
    for t in range(1, _NW):
        counts = counts + parts_ref[t]
    joint = counts * (1.0 / _N)
    mg = jnp.sum(joint, axis=1, keepdims=True)
    mi = jnp.sum(joint, axis=0, keepdims=True)
    valid = counts > 0.0
    denom = jnp.where(valid, mg * mi, 1.0)
    ratio = jnp.where(valid, joint / denom, 1.0)
    terms = jnp.where(valid, joint * jnp.log(ratio), 0.0)
    out_ref[...] = jnp.sum(counts * terms, keepdims=True)


_mi_call = pl.pallas_call(
    _mi_body,
    out_shape=jax.ShapeDtypeStruct((1, 1), jnp.float32),
)


def kernel(global_vector, infered_vector):
    parts = _joint_hist(global_vector, infered_vector)
    mi = _mi_call(parts)
    return mi[0, 0]
